# Initial kernel scaffold; baseline (speedup 1.0000x reference)
import jax
import jax.numpy as jnp
from jax.experimental import pallas as pl


def kernel(q_len, k_len, bias_table):
    raise NotImplementedError("write your pallas kernel here")


# speedup vs baseline: 51.2656x; 51.2656x over previous
import functools
import math

import jax
import jax.numpy as jnp
from jax import lax
from jax.experimental import pallas as pl
from jax.experimental.pallas import tpu as pltpu
from jax.experimental.pallas import tpu_sc as plsc

N_HEAD = 12
Q_LEN = 2048
K_LEN = 2048
LINE_LEN = Q_LEN + K_LEN - 1
ROW_PAD = 4232
N_SHIFT = 8


def _line8_body(shift_ref, table_ref, out_ref):
    h = pl.program_id(0)
    r = lax.broadcasted_iota(jnp.int32, (N_SHIFT, ROW_PAD), 0)
    j = lax.broadcasted_iota(jnp.int32, (N_SHIFT, ROW_PAD), 1)
    m = jnp.minimum(r + j, LINE_LEN - 1)
    d_tri = m - (Q_LEN - 1)
    rp = jnp.abs(d_tri + shift_ref[0, 0])
    rp_f = rp.astype(jnp.float32)
    t = jnp.log(rp_f / 8) / math.log(128 / 8) * (16 - 8)
    large = jnp.minimum(8 + t.astype(jnp.int32), 15)
    bucket = jnp.where(rp < 8, rp, large) + jnp.where(d_tri >= 1, 16, 0)
    acc = jnp.zeros((N_SHIFT, ROW_PAD), jnp.float32)
    for b in range(32):
        acc = jnp.where(bucket == b, table_ref[b, h], acc)
    out_ref[0] = acc


def _make_line8(shift, bias_table):
    return pl.pallas_call(
        _line8_body,
        grid=(N_HEAD,),
        in_specs=[
            pl.BlockSpec(memory_space=pltpu.SMEM),
            pl.BlockSpec(memory_space=pltpu.SMEM),
        ],
        out_specs=pl.BlockSpec((1, N_SHIFT, ROW_PAD), lambda h: (h, 0, 0)),
        out_shape=jax.ShapeDtypeStruct((N_HEAD, N_SHIFT, ROW_PAD), jnp.float32),
    )(shift, bias_table)


_HALF = Q_LEN // 2
_CHUNK = 16


@functools.cache
def _build_sc_expand():
    mesh = plsc.VectorSubcoreMesh(core_axis_name="c", subcore_axis_name="s")

    @functools.partial(
        pl.kernel,
        mesh=mesh,
        out_type=jax.ShapeDtypeStruct((N_HEAD * Q_LEN * K_LEN,), jnp.float32),
        scratch_types=[
            pltpu.VMEM((N_SHIFT * ROW_PAD,), jnp.float32),
            pltpu.SemaphoreType.DMA,
        ],
    )
    def _sc_expand(line8_hbm, out_hbm, line_v, sem):
        wid = lax.axis_index("s") * 2 + lax.axis_index("c")

        @pl.when(wid < N_HEAD * 2)
        def _():
            h = wid // 2
            q0 = (wid % 2) * _HALF
            pltpu.sync_copy(line8_hbm.at[h], line_v)

            def chunk(i, carry):
                qb = q0 + i * _CHUNK
                handles = []
                for jj in range(_CHUNK):
                    q = qb + jj
                    st = (Q_LEN - 1) - q
                    r = lax.rem(st, 8)
                    srcoff = pl.multiple_of(r * ROW_PAD + (st - r), 8)
                    dstoff = pl.multiple_of((h * Q_LEN + q) * K_LEN, 8)
                    handles.append(
                        pltpu.async_copy(
                            line_v.at[pl.ds(srcoff, K_LEN)],
                            out_hbm.at[pl.ds(dstoff, K_LEN)],
                            sem,
                        )
                    )
                for hd in handles:
                    hd.wait()
                return carry

            lax.fori_loop(0, _HALF // _CHUNK, chunk, 0)

    return _sc_expand


def kernel(q_len, k_len, bias_table):
    shift = jnp.asarray(k_len - q_len, jnp.int32).reshape(1, 1)
    line8 = _make_line8(shift, bias_table)
    flat = _build_sc_expand()(line8.reshape(N_HEAD, N_SHIFT * ROW_PAD))
    return flat.reshape(N_HEAD, Q_LEN, K_LEN)[None]


# speedup vs baseline: 53.7626x; 1.0487x over previous
import functools
import math

import jax
import jax.numpy as jnp
from jax import lax
from jax.experimental import pallas as pl
from jax.experimental.pallas import tpu as pltpu
from jax.experimental.pallas import tpu_sc as plsc

N_HEAD = 12
Q_LEN = 2048
K_LEN = 2048
LINE_LEN = Q_LEN + K_LEN - 1
ROW_PAD = 4232
N_SHIFT = 8


def _line8_body(shift_ref, table_ref, out_ref):
    h = pl.program_id(0)
    r = lax.broadcasted_iota(jnp.int32, (N_SHIFT, ROW_PAD), 0)
    j = lax.broadcasted_iota(jnp.int32, (N_SHIFT, ROW_PAD), 1)
    m = jnp.minimum(r + j, LINE_LEN - 1)
    d_tri = m - (Q_LEN - 1)
    rp = jnp.abs(d_tri + shift_ref[0, 0])
    rp_f = rp.astype(jnp.float32)
    t = jnp.log(rp_f / 8) / math.log(128 / 8) * (16 - 8)
    large = jnp.minimum(8 + t.astype(jnp.int32), 15)
    bucket = jnp.where(rp < 8, rp, large) + jnp.where(d_tri >= 1, 16, 0)
    acc = jnp.zeros((N_SHIFT, ROW_PAD), jnp.float32)
    for b in range(32):
        acc = jnp.where(bucket == b, table_ref[b, h], acc)
    out_ref[0] = acc


def _make_line8(shift, bias_table):
    return pl.pallas_call(
        _line8_body,
        grid=(N_HEAD,),
        in_specs=[
            pl.BlockSpec(memory_space=pltpu.SMEM),
            pl.BlockSpec(memory_space=pltpu.SMEM),
        ],
        out_specs=pl.BlockSpec((1, N_SHIFT, ROW_PAD), lambda h: (h, 0, 0)),
        out_shape=jax.ShapeDtypeStruct((N_HEAD, N_SHIFT, ROW_PAD), jnp.float32),
    )(shift, bias_table)


_N_WORKER = 32
_RPW = N_HEAD * Q_LEN // _N_WORKER
_CHUNK = 16


@functools.cache
def _build_sc_expand():
    mesh = plsc.VectorSubcoreMesh(core_axis_name="c", subcore_axis_name="s")

    @functools.partial(
        pl.kernel,
        mesh=mesh,
        out_type=jax.ShapeDtypeStruct((N_HEAD * Q_LEN * K_LEN,), jnp.float32),
        scratch_types=[
            pltpu.VMEM((2 * N_SHIFT * ROW_PAD,), jnp.float32),
            pltpu.SemaphoreType.DMA,
        ],
    )
    def _sc_expand(line8_hbm, out_hbm, line_v, sem):
        wid = lax.axis_index("s") * 2 + lax.axis_index("c")
        row0 = wid * _RPW
        h0 = row0 // Q_LEN
        h1 = (row0 + _RPW - 1) // Q_LEN
        line_words = N_SHIFT * ROW_PAD
        pltpu.sync_copy(
            line8_hbm.at[pl.ds(pl.multiple_of(h0 * line_words, 8), line_words)],
            line_v.at[pl.ds(0, line_words)])
        pltpu.sync_copy(
            line8_hbm.at[pl.ds(pl.multiple_of(h1 * line_words, 8), line_words)],
            line_v.at[pl.ds(line_words, line_words)])

        def fire(ci):
            handles = []
            for jj in range(_CHUNK):
                row = row0 + ci * _CHUNK + jj
                h = row // Q_LEN
                st = (Q_LEN - 1) - (row - h * Q_LEN)
                r = lax.rem(st, 8)
                srcoff = pl.multiple_of(
                    (h - h0) * line_words + r * ROW_PAD + (st - r), 8)
                dstoff = pl.multiple_of(row * K_LEN, 8)
                handles.append(
                    pltpu.async_copy(
                        line_v.at[pl.ds(srcoff, K_LEN)],
                        out_hbm.at[pl.ds(dstoff, K_LEN)],
                        sem,
                    )
                )
            return handles

        fire(0)

        def step(i, carry):
            for hd in fire(i + 1):
                hd.wait()
            return carry

        lax.fori_loop(0, _RPW // _CHUNK - 1, step, 0)
        for _ in range(_CHUNK):
            pltpu.make_async_copy(
                line_v.at[pl.ds(0, K_LEN)],
                out_hbm.at[pl.ds(0, K_LEN)],
                sem,
            ).wait()

    return _sc_expand


def kernel(q_len, k_len, bias_table):
    shift = jnp.asarray(k_len - q_len, jnp.int32).reshape(1, 1)
    line8 = _make_line8(shift, bias_table)
    flat = _build_sc_expand()(line8.reshape(N_HEAD * N_SHIFT * ROW_PAD))
    return flat.reshape(N_HEAD, Q_LEN, K_LEN)[None]
